# baseline (speedup 1.0000x reference)
import functools

import jax
import jax.numpy as jnp
from jax import lax
from jax.experimental import pallas as pl
from jax.experimental.pallas import tpu as pltpu
from jax.experimental.pallas import tpu_sc as plsc

RPS = 2


@functools.lru_cache(maxsize=None)
def _make_gather(V, D, B, S):
    info = plsc.get_sparse_core_info()
    NC, NS = info.num_cores, info.num_subcores
    NW = NC * NS
    rows_w = B // NW
    n_steps = rows_w // RPS
    assert B % NW == 0 and n_steps % 2 == 0
    splits = []
    off = 0
    while off < S:
        n = min(128, S - off)
        splits.append((off, n))
        off += (n + 7) // 8 * 8
    assert sum(n for _, n in splits) == S

    mesh = plsc.VectorSubcoreMesh(core_axis_name="c", subcore_axis_name="s")

    @functools.partial(
        pl.kernel,
        mesh=mesh,
        out_type=jax.ShapeDtypeStruct((B, S, D), jnp.float32),
        compiler_params=pltpu.CompilerParams(use_tc_tiling_on_sc=False),
        scratch_types=[
            pltpu.VMEM((rows_w, S), jnp.int32),
            pltpu.VMEM((2, RPS, S, D), jnp.float32),
            pltpu.SemaphoreType.DMA,
            pltpu.SemaphoreType.DMA,
        ],
    )
    def k(table_hbm, idx_hbm, out_hbm, idx_v, rows_v, sem_g, sem_w):
        wid = lax.axis_index("s") * NC + lax.axis_index("c")
        base = wid * rows_w

        pltpu.sync_copy(idx_hbm.at[pl.ds(base, rows_w)], idx_v)

        def fire_gathers(s, buf):
            for r in range(RPS):
                for off, n in splits:
                    pltpu.async_copy(
                        table_hbm.at[idx_v.at[s * RPS + r, pl.ds(off, n)]],
                        buf.at[r, pl.ds(off, n), :],
                        sem_g,
                    )

        def drain_gathers(buf):
            pltpu.make_async_copy(
                out_hbm.at[pl.ds(0, RPS)], buf, sem_g
            ).wait()

        def fire_writeback(s, buf):
            pltpu.async_copy(
                buf, out_hbm.at[pl.ds(base + s * RPS, RPS)], sem_w
            )

        def drain_writeback(buf):
            pltpu.make_async_copy(
                buf, out_hbm.at[pl.ds(0, RPS)], sem_w
            ).wait()

        buf0 = rows_v.at[0]
        buf1 = rows_v.at[1]

        fire_gathers(0, buf0)

        def body(g, carry):
            s0 = 2 * g

            @pl.when(g > 0)
            def _():
                drain_writeback(buf1)

            fire_gathers(s0 + 1, buf1)
            drain_gathers(buf0)
            fire_writeback(s0, buf0)

            drain_writeback(buf0)

            @pl.when(g < n_steps // 2 - 1)
            def _():
                fire_gathers(s0 + 2, buf0)

            drain_gathers(buf1)
            fire_writeback(s0 + 1, buf1)
            return carry

        lax.fori_loop(0, n_steps // 2, body, 0)
        drain_writeback(buf1)

    return k


def kernel(x, table):
    B, S = x.shape
    V, D = table.shape
    return _make_gather(V, D, B, S)(table, x.astype(jnp.int32))
